# baseline (speedup 1.0000x reference)
import jax
import jax.numpy as jnp
from jax import lax
from jax.experimental import pallas as pl
from jax.experimental.pallas import tpu as pltpu

V, D, B = 1024, 1024, 4096 * 50
M = 512

def _tc_body(idx_ref, hi_ref, lo_ref, out_ref):
    idxv = idx_ref[0, 0, :]
    iota = lax.broadcasted_iota(jnp.int32, (M, V), 1)
    oh = (idxv[:, None] == iota).astype(jnp.bfloat16)
    acc = jnp.dot(oh, hi_ref[...], preferred_element_type=jnp.float32)
    acc = acc + jnp.dot(oh, lo_ref[...], preferred_element_type=jnp.float32)
    out_ref[...] = acc

@jax.jit
def kernel(indices, emb_weight):
    idx = indices.reshape(B // M, 1, M).astype(jnp.int32)
    hi = emb_weight.astype(jnp.bfloat16)
    lo = (emb_weight - hi.astype(jnp.float32)).astype(jnp.bfloat16)
    out = pl.pallas_call(
        _tc_body,
        grid=(B // M,),
        in_specs=[
            pl.BlockSpec((1, 1, M), lambda i: (i, 0, 0)),
            pl.BlockSpec((V, D), lambda i: (0, 0)),
            pl.BlockSpec((V, D), lambda i: (0, 0)),
        ],
        out_specs=pl.BlockSpec((M, D), lambda i: (i, 0)),
        out_shape=jax.ShapeDtypeStruct((B, D), jnp.float32),
    )(idx, hi, lo)
    return out.reshape(4096, 50, D)
